# Initial kernel scaffold; baseline (speedup 1.0000x reference)
import jax
import jax.numpy as jnp
from jax.experimental import pallas as pl


def kernel(agent_feats, hideout_obs, timestep_obs, params, agent_batch):
    raise NotImplementedError("write your pallas kernel here")


# speedup vs baseline: 6.7153x; 6.7153x over previous
import functools

import jax
import jax.numpy as jnp
from jax.experimental import pallas as pl
from jax.experimental.pallas import tpu as pltpu

H = 32
NGATES = 4
GATES = ["i", "f", "c", "o"]
SMALL = ["agent_summ", "hideout_summ", "state_summ", "hideout", "timestep"]
BLK = {nt: i for i, nt in enumerate(SMALL)}
SW = H * len(SMALL)
SMALL_EDGES = [
    ("hideout", "hideout_summ"),
    ("hideout_summ", "state_summ"),
    ("agent_summ", "state_summ"),
    ("timestep", "state_summ"),
    ("hideout_summ", "hideout"),
    ("state_summ", "hideout_summ"),
    ("state_summ", "agent_summ"),
    ("state_summ", "timestep"),
]


def _ek(s, d):
    return s + "__" + d


def _pack_weights(params):
    Wx = jnp.concatenate([params[g]["W"]["agent"] for g in GATES], axis=1)
    Wr = jnp.concatenate(
        [params[g]["conv"][_ek("agent_summ", "agent")]["Wr"] for g in GATES], axis=1
    )
    Wlg = jnp.concatenate(
        [params[g]["conv"][_ek("agent_summ", "agent")]["Wl"] for g in GATES], axis=1
    )
    gb0 = jnp.concatenate(
        [
            (
                params[g]["conv"][_ek("agent_summ", "agent")]["bl"][None, :]
                + params[g]["b"]["agent"]
            )
            for g in GATES
        ],
        axis=1,
    )

    Mb0 = jnp.zeros((SW, NGATES * SW), jnp.float32)
    Mbm = jnp.zeros((H, NGATES * SW), jnp.float32)
    Mbx = jnp.zeros((2, NGATES * SW), jnp.float32)
    bxt = jnp.zeros((1, NGATES * SW), jnp.float32)
    brow = jnp.zeros((1, NGATES * SW), jnp.float32)

    for gi, g in enumerate(GATES):
        base = gi * SW
        for s, d in SMALL_EDGES:
            p = params[g]["conv"][_ek(s, d)]
            cs = base + BLK[d] * H
            Mb0 = Mb0.at[BLK[s] * H:(BLK[s] + 1) * H, cs:cs + H].add(p["Wl"])
            Mb0 = Mb0.at[BLK[d] * H:(BLK[d] + 1) * H, cs:cs + H].add(p["Wr"])
            brow = brow.at[0, cs:cs + H].add(p["bl"])
        p = params[g]["conv"][_ek("agent", "agent_summ")]
        cs = base + BLK["agent_summ"] * H
        Mbm = Mbm.at[:, cs:cs + H].add(p["Wl"])
        Mb0 = Mb0.at[BLK["agent_summ"] * H:(BLK["agent_summ"] + 1) * H,
                     cs:cs + H].add(p["Wr"])
        brow = brow.at[0, cs:cs + H].add(p["bl"])
        ch = base + BLK["hideout"] * H
        Mbx = Mbx.at[:, ch:ch + H].add(params[g]["W"]["hideout"])
        ct = base + BLK["timestep"] * H
        bxt = bxt.at[0, ct:ct + H].add(params[g]["W"]["timestep"][0])
        for nt in SMALL:
            cn = base + BLK[nt] * H
            brow = brow.at[0, cn:cn + H].add(params[g]["b"][nt][0])
    return Wx, Wr, Wlg, gb0, Mb0, Mbm, Mbx, bxt, brow


def _body(x_ref, abr_ref, abc_ref, xh_ref, xt_ref, Wx_ref, Wr_ref, Wlg_ref,
          gb0_ref, Mb0_ref, Mbm_ref, Mbx_ref, bxt_ref, brow_ref, out_ref,
          hc_a, h_s, c_s, m_s, cnt_s, gb_s, *, TILE, NTILES, TA, SEQ, NB):
    t = pl.program_id(0)
    j = pl.program_id(1)
    rows = pl.ds(j * TILE, TILE)

    @pl.when(t == 0)
    def _zero_tile():
        hc_a[rows, :] = jnp.zeros((TILE, 2 * H), jnp.float32)

    @pl.when((t == 0) & (j == 0))
    def _zero_small():
        h_s[...] = jnp.zeros((NB, SW), jnp.float32)
        c_s[...] = jnp.zeros((NB, SW), jnp.float32)
        cnt_s[...] = jnp.zeros((NB, 1), jnp.float32)

    @pl.when(j == 0)
    def _start_step():
        gb_s[...] = (
            jnp.dot(h_s[:, 0:H], Wlg_ref[...],
                    preferred_element_type=jnp.float32)
            + gb0_ref[...]
        )
        m_s[...] = jnp.zeros((NB, H), jnp.float32)

    ab_r = abr_ref[0]
    ab_c = abc_ref[0]
    oT = (jax.lax.broadcasted_iota(jnp.int32, (NB, TILE), 0) == ab_r
          ).astype(jnp.float32)
    oh = (jax.lax.broadcasted_iota(jnp.int32, (TILE, NB), 1) == ab_c
          ).astype(jnp.float32)

    @pl.when(t == 0)
    def _count():
        cnt_s[...] += jnp.dot(oT, jnp.ones((TILE, 1), jnp.float32),
                              preferred_element_type=jnp.float32)

    h_prev = hc_a[rows, 0:H]
    c_prev = hc_a[rows, H:2 * H]
    m_s[...] += jnp.dot(oT, h_prev, preferred_element_type=jnp.float32)

    x = x_ref[0]
    pre = (
        jnp.dot(x, Wx_ref[...], preferred_element_type=jnp.float32)
        + jnp.dot(h_prev, Wr_ref[...], preferred_element_type=jnp.float32)
        + jnp.dot(oh, gb_s[...], preferred_element_type=jnp.float32)
    )
    ig = jax.nn.sigmoid(pre[:, 0:H])
    fg = jax.nn.sigmoid(pre[:, H:2 * H])
    gg = jnp.tanh(pre[:, 2 * H:3 * H])
    og = jax.nn.sigmoid(pre[:, 3 * H:4 * H])
    c_new = fg * c_prev + ig * gg
    h_new = og * jnp.tanh(c_new)
    valid = (j * TILE + jax.lax.broadcasted_iota(jnp.int32, (TILE, 1), 0)) < TA
    hc_a[rows, :] = jnp.where(
        valid, jnp.concatenate([h_new, c_new], axis=1), 0.0)

    @pl.when(j == NTILES - 1)
    def _small_step():
        m = m_s[...] / jnp.maximum(cnt_s[...], 1.0)
        hs = h_s[...]
        pre_s = (
            jnp.dot(hs, Mb0_ref[...], preferred_element_type=jnp.float32)
            + jnp.dot(m, Mbm_ref[...], preferred_element_type=jnp.float32)
            + jnp.dot(xh_ref[...], Mbx_ref[...],
                      preferred_element_type=jnp.float32)
            + xt_ref[0] * bxt_ref[...]
            + brow_ref[...]
        )
        i_s = jax.nn.sigmoid(pre_s[:, 0:SW])
        f_s = jax.nn.sigmoid(pre_s[:, SW:2 * SW])
        g_s = jnp.tanh(pre_s[:, 2 * SW:3 * SW])
        o_s = jax.nn.sigmoid(pre_s[:, 3 * SW:4 * SW])
        c_ns = f_s * c_s[...] + i_s * g_s
        h_ns = o_s * jnp.tanh(c_ns)
        c_s[...] = c_ns
        h_s[...] = h_ns

        @pl.when(t == SEQ - 1)
        def _emit():
            out_ref[...] = h_ns[:, 2 * H:3 * H]


def kernel(agent_feats, hideout_obs, timestep_obs, params, agent_batch):
    SEQ, TA, F = agent_feats.shape
    NB = hideout_obs.shape[0]
    TILE = 1024
    NTILES = max(1, -(-TA // TILE))
    TAP = NTILES * TILE

    Wx, Wr, Wlg, gb0, Mb0, Mbm, Mbx, bxt, brow = _pack_weights(params)

    ab = agent_batch.astype(jnp.int32)
    abp = jnp.pad(ab, (0, TAP - TA), constant_values=127)
    ab_row = abp.reshape(NTILES, 1, TILE)
    ab_col = abp.reshape(NTILES, TILE, 1)
    xt = timestep_obs.T.reshape(SEQ, NB, 1)

    body = functools.partial(_body, TILE=TILE, NTILES=NTILES, TA=TA, SEQ=SEQ,
                             NB=NB)
    grid = (SEQ, NTILES)
    out = pl.pallas_call(
        body,
        grid=grid,
        in_specs=[
            pl.BlockSpec((1, TILE, F), lambda t, j: (t, j, 0)),
            pl.BlockSpec((1, 1, TILE), lambda t, j: (j, 0, 0)),
            pl.BlockSpec((1, TILE, 1), lambda t, j: (j, 0, 0)),
            pl.BlockSpec((NB, 2), lambda t, j: (0, 0)),
            pl.BlockSpec((1, NB, 1), lambda t, j: (t, 0, 0)),
            pl.BlockSpec((F, NGATES * H), lambda t, j: (0, 0)),
            pl.BlockSpec((H, NGATES * H), lambda t, j: (0, 0)),
            pl.BlockSpec((H, NGATES * H), lambda t, j: (0, 0)),
            pl.BlockSpec((1, NGATES * H), lambda t, j: (0, 0)),
            pl.BlockSpec((SW, NGATES * SW), lambda t, j: (0, 0)),
            pl.BlockSpec((H, NGATES * SW), lambda t, j: (0, 0)),
            pl.BlockSpec((2, NGATES * SW), lambda t, j: (0, 0)),
            pl.BlockSpec((1, NGATES * SW), lambda t, j: (0, 0)),
            pl.BlockSpec((1, NGATES * SW), lambda t, j: (0, 0)),
        ],
        out_specs=pl.BlockSpec((NB, H), lambda t, j: (0, 0)),
        out_shape=jax.ShapeDtypeStruct((NB, H), jnp.float32),
        scratch_shapes=[
            pltpu.VMEM((TAP, 2 * H), jnp.float32),
            pltpu.VMEM((NB, SW), jnp.float32),
            pltpu.VMEM((NB, SW), jnp.float32),
            pltpu.VMEM((NB, H), jnp.float32),
            pltpu.VMEM((NB, 1), jnp.float32),
            pltpu.VMEM((NB, NGATES * H), jnp.float32),
        ],
    )(agent_feats, ab_row, ab_col, hideout_obs, xt, Wx, Wr, Wlg, gb0,
      Mb0, Mbm, Mbx, bxt, brow)
    return out


# speedup vs baseline: 11.3362x; 1.6881x over previous
import functools

import jax
import jax.numpy as jnp
from jax.experimental import pallas as pl
from jax.experimental.pallas import tpu as pltpu

H = 32
NGATES = 4
GATES = ["i", "f", "c", "o"]
SMALL = ["agent_summ", "hideout_summ", "state_summ", "hideout", "timestep"]
BLK = {nt: i for i, nt in enumerate(SMALL)}
SW = H * len(SMALL)
SMALL_EDGES = [
    ("hideout", "hideout_summ"),
    ("hideout_summ", "state_summ"),
    ("agent_summ", "state_summ"),
    ("timestep", "state_summ"),
    ("hideout_summ", "hideout"),
    ("state_summ", "hideout_summ"),
    ("state_summ", "agent_summ"),
    ("state_summ", "timestep"),
]


def _ek(s, d):
    return s + "__" + d


def _pack_weights(params):
    WxT = jnp.concatenate(
        [params[g]["W"]["agent"].T for g in GATES], axis=0)
    WrT = jnp.concatenate(
        [params[g]["conv"][_ek("agent_summ", "agent")]["Wr"].T for g in GATES],
        axis=0)
    WlgT = jnp.concatenate(
        [params[g]["conv"][_ek("agent_summ", "agent")]["Wl"].T for g in GATES],
        axis=0)
    gb0T = jnp.concatenate(
        [
            (
                params[g]["conv"][_ek("agent_summ", "agent")]["bl"][:, None]
                + params[g]["b"]["agent"].T
            )
            for g in GATES
        ],
        axis=0,
    )

    Mb0 = jnp.zeros((SW, NGATES * SW), jnp.float32)
    Mbm = jnp.zeros((H, NGATES * SW), jnp.float32)
    Mbx = jnp.zeros((2, NGATES * SW), jnp.float32)
    bxt = jnp.zeros((1, NGATES * SW), jnp.float32)
    brow = jnp.zeros((1, NGATES * SW), jnp.float32)

    for gi, g in enumerate(GATES):
        base = gi * SW
        for s, d in SMALL_EDGES:
            p = params[g]["conv"][_ek(s, d)]
            cs = base + BLK[d] * H
            Mb0 = Mb0.at[BLK[s] * H:(BLK[s] + 1) * H, cs:cs + H].add(p["Wl"])
            Mb0 = Mb0.at[BLK[d] * H:(BLK[d] + 1) * H, cs:cs + H].add(p["Wr"])
            brow = brow.at[0, cs:cs + H].add(p["bl"])
        p = params[g]["conv"][_ek("agent", "agent_summ")]
        cs = base + BLK["agent_summ"] * H
        Mbm = Mbm.at[:, cs:cs + H].add(p["Wl"])
        Mb0 = Mb0.at[BLK["agent_summ"] * H:(BLK["agent_summ"] + 1) * H,
                     cs:cs + H].add(p["Wr"])
        brow = brow.at[0, cs:cs + H].add(p["bl"])
        ch = base + BLK["hideout"] * H
        Mbx = Mbx.at[:, ch:ch + H].add(params[g]["W"]["hideout"])
        ct = base + BLK["timestep"] * H
        bxt = bxt.at[0, ct:ct + H].add(params[g]["W"]["timestep"][0])
        for nt in SMALL:
            cn = base + BLK[nt] * H
            brow = brow.at[0, cn:cn + H].add(params[g]["b"][nt][0])
    return WxT, WrT, WlgT, gb0T, Mb0.T, Mbm.T, Mbx.T, bxt.T, brow.T


def _dot(a, b):
    return jnp.dot(a, b, preferred_element_type=jnp.float32)


def _body(x_ref, abr_ref, abc_ref, xh_ref, xt_ref, Wx_ref, Wr_ref, Wlg_ref,
          gb0_ref, Mb0_ref, Mbm_ref, Mbx_ref, bxt_ref, brow_ref, out_ref,
          hc_a, h_s, c_s, m_s, cnt_s, gb_s, *, TILE, NTILES, TA, SEQ, NB):
    t = pl.program_id(0)
    j = pl.program_id(1)
    cols = pl.ds(j * TILE, TILE)

    @pl.when(t == 0)
    def _zero_tile():
        hc_a[:, cols] = jnp.zeros((2 * H, TILE), jnp.float32)

    @pl.when((t == 0) & (j == 0))
    def _zero_small():
        h_s[...] = jnp.zeros((SW, NB), jnp.float32)
        c_s[...] = jnp.zeros((SW, NB), jnp.float32)
        cnt_s[...] = jnp.zeros((1, NB), jnp.float32)

    @pl.when(j == 0)
    def _start_step():
        gb_s[...] = _dot(Wlg_ref[...], h_s[0:H, :]) + gb0_ref[...]
        m_s[...] = jnp.zeros((H, NB), jnp.float32)

    ab_r = abr_ref[0]
    ab_c = abc_ref[0]
    oT = (jax.lax.broadcasted_iota(jnp.int32, (NB, TILE), 0) == ab_r
          ).astype(jnp.float32)
    oh = (jax.lax.broadcasted_iota(jnp.int32, (TILE, NB), 1) == ab_c
          ).astype(jnp.float32)

    @pl.when(t == 0)
    def _count():
        cnt_s[...] += _dot(jnp.ones((1, TILE), jnp.float32), oh)

    h_prev = hc_a[0:H, cols]
    c_prev = hc_a[H:2 * H, cols]
    m_s[...] += _dot(h_prev, oh)

    x = x_ref[0]
    pre = (_dot(Wx_ref[...], x) + _dot(Wr_ref[...], h_prev)
           + _dot(gb_s[...], oT))
    ig = jax.nn.sigmoid(pre[0:H, :])
    fg = jax.nn.sigmoid(pre[H:2 * H, :])
    gg = jnp.tanh(pre[2 * H:3 * H, :])
    og = jax.nn.sigmoid(pre[3 * H:4 * H, :])
    c_new = fg * c_prev + ig * gg
    h_new = og * jnp.tanh(c_new)
    valid = (j * TILE + jax.lax.broadcasted_iota(jnp.int32, (1, TILE), 1)) < TA
    hc_a[:, cols] = jnp.where(
        valid, jnp.concatenate([h_new, c_new], axis=0), 0.0)

    @pl.when(j == NTILES - 1)
    def _small_step():
        m = m_s[...] / jnp.maximum(cnt_s[...], 1.0)
        pre_s = (_dot(Mb0_ref[...], h_s[...]) + _dot(Mbm_ref[...], m)
                 + _dot(Mbx_ref[...], xh_ref[...])
                 + bxt_ref[...] * xt_ref[0]
                 + brow_ref[...])
        i_s = jax.nn.sigmoid(pre_s[0:SW, :])
        f_s = jax.nn.sigmoid(pre_s[SW:2 * SW, :])
        g_s = jnp.tanh(pre_s[2 * SW:3 * SW, :])
        o_s = jax.nn.sigmoid(pre_s[3 * SW:4 * SW, :])
        c_ns = f_s * c_s[...] + i_s * g_s
        h_ns = o_s * jnp.tanh(c_ns)
        c_s[...] = c_ns
        h_s[...] = h_ns

        @pl.when(t == SEQ - 1)
        def _emit():
            out_ref[...] = h_ns[2 * H:3 * H, :]


def kernel(agent_feats, hideout_obs, timestep_obs, params, agent_batch):
    SEQ, TA, F = agent_feats.shape
    NB = hideout_obs.shape[0]
    TILE = 2048
    NTILES = max(1, -(-TA // TILE))
    TAP = NTILES * TILE

    WxT, WrT, WlgT, gb0T, Mb0T, MbmT, MbxT, bxtT, browT = _pack_weights(params)

    ab = agent_batch.astype(jnp.int32)
    abp = jnp.pad(ab, (0, TAP - TA), constant_values=127)
    ab_row = abp.reshape(NTILES, 1, TILE)
    ab_col = abp.reshape(NTILES, TILE, 1)
    afT = agent_feats.transpose(0, 2, 1)
    xhT = hideout_obs.T
    xt3 = timestep_obs.T.reshape(SEQ, 1, NB)

    body = functools.partial(_body, TILE=TILE, NTILES=NTILES, TA=TA, SEQ=SEQ,
                             NB=NB)
    grid = (SEQ, NTILES)
    outT = pl.pallas_call(
        body,
        grid=grid,
        in_specs=[
            pl.BlockSpec((1, F, TILE), lambda t, j: (t, 0, j)),
            pl.BlockSpec((1, 1, TILE), lambda t, j: (j, 0, 0)),
            pl.BlockSpec((1, TILE, 1), lambda t, j: (j, 0, 0)),
            pl.BlockSpec((2, NB), lambda t, j: (0, 0)),
            pl.BlockSpec((1, 1, NB), lambda t, j: (t, 0, 0)),
            pl.BlockSpec((NGATES * H, F), lambda t, j: (0, 0)),
            pl.BlockSpec((NGATES * H, H), lambda t, j: (0, 0)),
            pl.BlockSpec((NGATES * H, H), lambda t, j: (0, 0)),
            pl.BlockSpec((NGATES * H, 1), lambda t, j: (0, 0)),
            pl.BlockSpec((NGATES * SW, SW), lambda t, j: (0, 0)),
            pl.BlockSpec((NGATES * SW, H), lambda t, j: (0, 0)),
            pl.BlockSpec((NGATES * SW, 2), lambda t, j: (0, 0)),
            pl.BlockSpec((NGATES * SW, 1), lambda t, j: (0, 0)),
            pl.BlockSpec((NGATES * SW, 1), lambda t, j: (0, 0)),
        ],
        out_specs=pl.BlockSpec((H, NB), lambda t, j: (0, 0)),
        out_shape=jax.ShapeDtypeStruct((H, NB), jnp.float32),
        scratch_shapes=[
            pltpu.VMEM((2 * H, TAP), jnp.float32),
            pltpu.VMEM((SW, NB), jnp.float32),
            pltpu.VMEM((SW, NB), jnp.float32),
            pltpu.VMEM((H, NB), jnp.float32),
            pltpu.VMEM((1, NB), jnp.float32),
            pltpu.VMEM((NGATES * H, NB), jnp.float32),
        ],
    )(afT, ab_row, ab_col, xhT, xt3, WxT, WrT, WlgT, gb0T,
      Mb0T, MbmT, MbxT, bxtT, browT)
    return outT.T


# speedup vs baseline: 12.1912x; 1.0754x over previous
import functools

import jax
import jax.numpy as jnp
from jax.experimental import pallas as pl
from jax.experimental.pallas import tpu as pltpu

H = 32
NGATES = 4
GATES = ["i", "f", "c", "o"]
SMALL = ["agent_summ", "hideout_summ", "state_summ", "hideout", "timestep"]
BLK = {nt: i for i, nt in enumerate(SMALL)}
SW = H * len(SMALL)
SMALL_EDGES = [
    ("hideout", "hideout_summ"),
    ("hideout_summ", "state_summ"),
    ("agent_summ", "state_summ"),
    ("timestep", "state_summ"),
    ("hideout_summ", "hideout"),
    ("state_summ", "hideout_summ"),
    ("state_summ", "agent_summ"),
    ("state_summ", "timestep"),
]


def _ek(s, d):
    return s + "__" + d


def _pack_weights(params):
    WxT = jnp.concatenate(
        [params[g]["W"]["agent"].T for g in GATES], axis=0)
    WrT = jnp.concatenate(
        [params[g]["conv"][_ek("agent_summ", "agent")]["Wr"].T for g in GATES],
        axis=0)
    WlgT = jnp.concatenate(
        [params[g]["conv"][_ek("agent_summ", "agent")]["Wl"].T for g in GATES],
        axis=0)
    gb0T = jnp.concatenate(
        [
            (
                params[g]["conv"][_ek("agent_summ", "agent")]["bl"][:, None]
                + params[g]["b"]["agent"].T
            )
            for g in GATES
        ],
        axis=0,
    )

    Mb0 = jnp.zeros((SW, NGATES * SW), jnp.float32)
    Mbm = jnp.zeros((H, NGATES * SW), jnp.float32)
    Mbx = jnp.zeros((2, NGATES * SW), jnp.float32)
    bxt = jnp.zeros((1, NGATES * SW), jnp.float32)
    brow = jnp.zeros((1, NGATES * SW), jnp.float32)

    for gi, g in enumerate(GATES):
        base = gi * SW
        for s, d in SMALL_EDGES:
            p = params[g]["conv"][_ek(s, d)]
            cs = base + BLK[d] * H
            Mb0 = Mb0.at[BLK[s] * H:(BLK[s] + 1) * H, cs:cs + H].add(p["Wl"])
            Mb0 = Mb0.at[BLK[d] * H:(BLK[d] + 1) * H, cs:cs + H].add(p["Wr"])
            brow = brow.at[0, cs:cs + H].add(p["bl"])
        p = params[g]["conv"][_ek("agent", "agent_summ")]
        cs = base + BLK["agent_summ"] * H
        Mbm = Mbm.at[:, cs:cs + H].add(p["Wl"])
        Mb0 = Mb0.at[BLK["agent_summ"] * H:(BLK["agent_summ"] + 1) * H,
                     cs:cs + H].add(p["Wr"])
        brow = brow.at[0, cs:cs + H].add(p["bl"])
        ch = base + BLK["hideout"] * H
        Mbx = Mbx.at[:, ch:ch + H].add(params[g]["W"]["hideout"])
        ct = base + BLK["timestep"] * H
        bxt = bxt.at[0, ct:ct + H].add(params[g]["W"]["timestep"][0])
        for nt in SMALL:
            cn = base + BLK[nt] * H
            brow = brow.at[0, cn:cn + H].add(params[g]["b"][nt][0])
    return WxT, WrT, WlgT, gb0T, Mb0.T, Mbm.T, Mbx.T, bxt.T, brow.T


def _dot(a, b):
    return jnp.dot(a, b, preferred_element_type=jnp.float32)


BF = jnp.bfloat16


def _body(x_ref, abr_ref, abc_ref, xh_ref, xt_ref, Wx_ref, Wr_ref, Wlg_ref,
          gb0_ref, Mb0_ref, Mbm_ref, Mbx_ref, bxt_ref, brow_ref, out_ref,
          h_a, c_a, h_s, c_s, m_s, gb_s, *, TILE, NTILES, TA, SEQ, NB):
    t = pl.program_id(0)
    j = pl.program_id(1)
    cols = pl.ds(j * TILE, TILE)

    @pl.when(t == 0)
    def _zero_tile():
        h_a[:, cols] = jnp.concatenate(
            [jnp.zeros((H, TILE), BF), jnp.ones((8, TILE), BF)], axis=0)
        c_a[:, cols] = jnp.zeros((H, TILE), jnp.float32)

    @pl.when((t == 0) & (j == 0))
    def _zero_small():
        h_s[...] = jnp.zeros((SW, NB), jnp.float32)
        c_s[...] = jnp.zeros((SW, NB), jnp.float32)

    @pl.when(j == 0)
    def _start_step():
        gb_s[...] = (_dot(Wlg_ref[...], h_s[0:H, :])
                     + gb0_ref[...]).astype(BF)
        m_s[...] = jnp.zeros((H + 8, NB), jnp.float32)

    ab_r = abr_ref[0]
    ab_c = abc_ref[0]
    oT = (jax.lax.broadcasted_iota(jnp.int32, (NB, TILE), 0) == ab_r
          ).astype(BF)
    oh = (jax.lax.broadcasted_iota(jnp.int32, (TILE, NB), 1) == ab_c
          ).astype(BF)

    h1_prev = h_a[:, cols]
    c_prev = c_a[:, cols]
    m_s[...] += _dot(h1_prev, oh)

    x = x_ref[0]
    pre = (_dot(Wx_ref[...], x) + _dot(Wr_ref[...], h1_prev[0:H, :])
           + _dot(gb_s[...], oT))
    ig = jax.nn.sigmoid(pre[0:H, :])
    fg = jax.nn.sigmoid(pre[H:2 * H, :])
    gg = jnp.tanh(pre[2 * H:3 * H, :])
    og = jax.nn.sigmoid(pre[3 * H:4 * H, :])
    c_new = fg * c_prev + ig * gg
    h_new = og * jnp.tanh(c_new)
    valid = (j * TILE + jax.lax.broadcasted_iota(jnp.int32, (1, TILE), 1)) < TA
    h_a[0:H, cols] = jnp.where(valid, h_new, 0.0).astype(BF)
    c_a[:, cols] = jnp.where(valid, c_new, 0.0)

    @pl.when(j == NTILES - 1)
    def _small_step():
        m = m_s[0:H, :] / jnp.maximum(m_s[H:H + 1, :], 1.0)
        pre_s = (_dot(Mb0_ref[...], h_s[...]) + _dot(Mbm_ref[...], m)
                 + _dot(Mbx_ref[...], xh_ref[...])
                 + bxt_ref[...] * xt_ref[0]
                 + brow_ref[...])
        i_s = jax.nn.sigmoid(pre_s[0:SW, :])
        f_s = jax.nn.sigmoid(pre_s[SW:2 * SW, :])
        g_s = jnp.tanh(pre_s[2 * SW:3 * SW, :])
        o_s = jax.nn.sigmoid(pre_s[3 * SW:4 * SW, :])
        c_ns = f_s * c_s[...] + i_s * g_s
        h_ns = o_s * jnp.tanh(c_ns)
        c_s[...] = c_ns
        h_s[...] = h_ns

        @pl.when(t == SEQ - 1)
        def _emit():
            out_ref[...] = h_ns[2 * H:3 * H, :]


def kernel(agent_feats, hideout_obs, timestep_obs, params, agent_batch):
    SEQ, TA, F = agent_feats.shape
    NB = hideout_obs.shape[0]
    TILE = 4096
    NTILES = max(1, -(-TA // TILE))
    TAP = NTILES * TILE

    WxT, WrT, WlgT, gb0T, Mb0T, MbmT, MbxT, bxtT, browT = _pack_weights(params)

    ab = agent_batch.astype(jnp.int32)
    abp = jnp.pad(ab, (0, TAP - TA), constant_values=127)
    ab_row = abp.reshape(NTILES, 1, TILE)
    ab_col = abp.reshape(NTILES, TILE, 1)
    afT = agent_feats.transpose(0, 2, 1).astype(BF)
    xhT = hideout_obs.T
    xt3 = timestep_obs.T.reshape(SEQ, 1, NB)

    body = functools.partial(_body, TILE=TILE, NTILES=NTILES, TA=TA, SEQ=SEQ,
                             NB=NB)
    grid = (SEQ, NTILES)
    outT = pl.pallas_call(
        body,
        grid=grid,
        in_specs=[
            pl.BlockSpec((1, F, TILE), lambda t, j: (t, 0, j)),
            pl.BlockSpec((1, 1, TILE), lambda t, j: (j, 0, 0)),
            pl.BlockSpec((1, TILE, 1), lambda t, j: (j, 0, 0)),
            pl.BlockSpec((2, NB), lambda t, j: (0, 0)),
            pl.BlockSpec((1, 1, NB), lambda t, j: (t, 0, 0)),
            pl.BlockSpec((NGATES * H, F), lambda t, j: (0, 0)),
            pl.BlockSpec((NGATES * H, H), lambda t, j: (0, 0)),
            pl.BlockSpec((NGATES * H, H), lambda t, j: (0, 0)),
            pl.BlockSpec((NGATES * H, 1), lambda t, j: (0, 0)),
            pl.BlockSpec((NGATES * SW, SW), lambda t, j: (0, 0)),
            pl.BlockSpec((NGATES * SW, H), lambda t, j: (0, 0)),
            pl.BlockSpec((NGATES * SW, 2), lambda t, j: (0, 0)),
            pl.BlockSpec((NGATES * SW, 1), lambda t, j: (0, 0)),
            pl.BlockSpec((NGATES * SW, 1), lambda t, j: (0, 0)),
        ],
        out_specs=pl.BlockSpec((H, NB), lambda t, j: (0, 0)),
        out_shape=jax.ShapeDtypeStruct((H, NB), jnp.float32),
        scratch_shapes=[
            pltpu.VMEM((H + 8, TAP), BF),
            pltpu.VMEM((H, TAP), jnp.float32),
            pltpu.VMEM((SW, NB), jnp.float32),
            pltpu.VMEM((SW, NB), jnp.float32),
            pltpu.VMEM((H + 8, NB), jnp.float32),
            pltpu.VMEM((NGATES * H, NB), BF),
        ],
    )(afT, ab_row, ab_col, xhT, xt3, WxT.astype(BF), WrT.astype(BF), WlgT,
      gb0T, Mb0T, MbmT, MbxT, bxtT, browT)
    return outT.T


# speedup vs baseline: 21.4273x; 1.7576x over previous
import functools

import jax
import jax.numpy as jnp
from jax.experimental import pallas as pl
from jax.experimental.pallas import tpu as pltpu

H = 32
NGATES = 4
GATES = ["i", "f", "c", "o"]
SMALL = ["agent_summ", "hideout_summ", "state_summ", "hideout", "timestep"]
BLK = {nt: i for i, nt in enumerate(SMALL)}
SW = H * len(SMALL)
SMALL_EDGES = [
    ("hideout", "hideout_summ"),
    ("hideout_summ", "state_summ"),
    ("agent_summ", "state_summ"),
    ("timestep", "state_summ"),
    ("hideout_summ", "hideout"),
    ("state_summ", "hideout_summ"),
    ("state_summ", "agent_summ"),
    ("state_summ", "timestep"),
]


def _ek(s, d):
    return s + "__" + d


def _pack_weights(params):
    WxT = jnp.concatenate(
        [params[g]["W"]["agent"] for g in GATES], axis=1).T
    WrT = jnp.concatenate(
        [params[g]["conv"][_ek("agent_summ", "agent")]["Wr"] for g in GATES],
        axis=1).T
    WlgT = jnp.concatenate(
        [params[g]["conv"][_ek("agent_summ", "agent")]["Wl"] for g in GATES],
        axis=1).T
    gb0T = jnp.concatenate(
        [
            (
                params[g]["conv"][_ek("agent_summ", "agent")]["bl"][None, :]
                + params[g]["b"]["agent"]
            )
            for g in GATES
        ],
        axis=1,
    ).T

    ZH = jnp.zeros((H, H), jnp.float32)
    rows = []
    for s in SMALL:
        row = []
        for g in GATES:
            conv = params[g]["conv"]
            for d in SMALL:
                acc = ZH
                for es, ed in SMALL_EDGES:
                    if ed != d:
                        continue
                    p = conv[_ek(es, ed)]
                    if es == s:
                        acc = acc + p["Wl"]
                    if ed == s:
                        acc = acc + p["Wr"]
                if d == "agent_summ" and s == "agent_summ":
                    acc = acc + conv[_ek("agent", "agent_summ")]["Wr"]
                row.append(acc)
        rows.append(jnp.concatenate(row, axis=1))
    Mb0 = jnp.concatenate(rows, axis=0)

    Z2 = jnp.zeros((2, H), jnp.float32)
    Z1 = jnp.zeros((1, H), jnp.float32)
    mbm, mbx, bxt_r, brow_r = [], [], [], []
    for g in GATES:
        conv = params[g]["conv"]
        for d in SMALL:
            mbm.append(conv[_ek("agent", "agent_summ")]["Wl"]
                       if d == "agent_summ" else ZH)
            mbx.append(params[g]["W"]["hideout"] if d == "hideout" else Z2)
            bxt_r.append(params[g]["W"]["timestep"]
                         if d == "timestep" else Z1)
            b = params[g]["b"][d]
            for es, ed in SMALL_EDGES:
                if ed == d:
                    b = b + conv[_ek(es, ed)]["bl"][None, :]
            if d == "agent_summ":
                b = b + conv[_ek("agent", "agent_summ")]["bl"][None, :]
            brow_r.append(b)
    Mbm = jnp.concatenate(mbm, axis=1)
    Mbx = jnp.concatenate(mbx, axis=1)
    bxt = jnp.concatenate(bxt_r, axis=1)
    brow = jnp.concatenate(brow_r, axis=1)
    return WxT, WrT, WlgT, gb0T, Mb0.T, Mbm.T, Mbx.T, bxt.T, brow.T


def _dot(a, b):
    return jnp.dot(a, b, preferred_element_type=jnp.float32)


BF = jnp.bfloat16


def _body(x_ref, abr_ref, abc_ref, xh_ref, xt_ref, Wx_ref, Wr_ref, Wlg_ref,
          gb0_ref, Mb0_ref, Mbm_ref, Mbx_ref, bxt_ref, brow_ref, out_ref,
          h_a, c_a, h_s, c_s, m_s, gb_s, *, TILE, NTILES, TA, SEQ, NB):
    t = pl.program_id(0)
    j = pl.program_id(1)
    cols = pl.ds(j * TILE, TILE)

    @pl.when(t == 0)
    def _zero_tile():
        h_a[:, cols] = jnp.concatenate(
            [jnp.zeros((H, TILE), BF), jnp.ones((8, TILE), BF)], axis=0)
        c_a[:, cols] = jnp.zeros((H, TILE), jnp.float32)

    @pl.when((t == 0) & (j == 0))
    def _zero_small():
        h_s[...] = jnp.zeros((SW, NB), jnp.float32)
        c_s[...] = jnp.zeros((SW, NB), jnp.float32)

    @pl.when(j == 0)
    def _start_step():
        gb_s[...] = (_dot(Wlg_ref[...], h_s[0:H, :])
                     + gb0_ref[...]).astype(BF)
        m_s[...] = jnp.zeros((H + 8, NB), jnp.float32)

    ab_r = abr_ref[0]
    ab_c = abc_ref[0]
    oT = (jax.lax.broadcasted_iota(jnp.int32, (NB, TILE), 0) == ab_r
          ).astype(BF)
    oh = (jax.lax.broadcasted_iota(jnp.int32, (TILE, NB), 1) == ab_c
          ).astype(BF)

    h1_prev = h_a[:, cols]
    c_prev = c_a[:, cols]
    m_s[...] += _dot(h1_prev, oh)

    x = x_ref[0]
    pre = (_dot(Wx_ref[...], x) + _dot(Wr_ref[...], h1_prev[0:H, :])
           + _dot(gb_s[...], oT))
    ig = jax.nn.sigmoid(pre[0:H, :])
    fg = jax.nn.sigmoid(pre[H:2 * H, :])
    gg = jnp.tanh(pre[2 * H:3 * H, :])
    og = jax.nn.sigmoid(pre[3 * H:4 * H, :])
    c_new = fg * c_prev + ig * gg
    h_new = og * jnp.tanh(c_new)
    valid = (j * TILE + jax.lax.broadcasted_iota(jnp.int32, (1, TILE), 1)) < TA
    h_a[0:H, cols] = jnp.where(valid, h_new, 0.0).astype(BF)
    c_a[:, cols] = jnp.where(valid, c_new, 0.0)

    @pl.when(j == NTILES - 1)
    def _small_step():
        m = m_s[0:H, :] / jnp.maximum(m_s[H:H + 1, :], 1.0)
        pre_s = (_dot(Mb0_ref[...], h_s[...]) + _dot(Mbm_ref[...], m)
                 + _dot(Mbx_ref[...], xh_ref[...])
                 + bxt_ref[...] * xt_ref[0]
                 + brow_ref[...])
        i_s = jax.nn.sigmoid(pre_s[0:SW, :])
        f_s = jax.nn.sigmoid(pre_s[SW:2 * SW, :])
        g_s = jnp.tanh(pre_s[2 * SW:3 * SW, :])
        o_s = jax.nn.sigmoid(pre_s[3 * SW:4 * SW, :])
        c_ns = f_s * c_s[...] + i_s * g_s
        h_ns = o_s * jnp.tanh(c_ns)
        c_s[...] = c_ns
        h_s[...] = h_ns

        @pl.when(t == SEQ - 1)
        def _emit():
            out_ref[...] = h_ns[2 * H:3 * H, :]


def kernel(agent_feats, hideout_obs, timestep_obs, params, agent_batch):
    SEQ, TA, F = agent_feats.shape
    NB = hideout_obs.shape[0]
    TILE = 4096
    NTILES = max(1, -(-TA // TILE))
    TAP = NTILES * TILE

    WxT, WrT, WlgT, gb0T, Mb0T, MbmT, MbxT, bxtT, browT = _pack_weights(params)

    ab = agent_batch.astype(jnp.int32)
    abp = jnp.pad(ab, (0, TAP - TA), constant_values=127)
    ab_row = abp.reshape(NTILES, 1, TILE)
    ab_col = abp.reshape(NTILES, TILE, 1)
    afT = agent_feats.transpose(0, 2, 1).astype(BF)
    xhT = hideout_obs.T
    xt3 = timestep_obs.T.reshape(SEQ, 1, NB)

    body = functools.partial(_body, TILE=TILE, NTILES=NTILES, TA=TA, SEQ=SEQ,
                             NB=NB)
    grid = (SEQ, NTILES)
    outT = pl.pallas_call(
        body,
        grid=grid,
        in_specs=[
            pl.BlockSpec((1, F, TILE), lambda t, j: (t, 0, j)),
            pl.BlockSpec((1, 1, TILE), lambda t, j: (j, 0, 0)),
            pl.BlockSpec((1, TILE, 1), lambda t, j: (j, 0, 0)),
            pl.BlockSpec((2, NB), lambda t, j: (0, 0)),
            pl.BlockSpec((1, 1, NB), lambda t, j: (t, 0, 0)),
            pl.BlockSpec((NGATES * H, F), lambda t, j: (0, 0)),
            pl.BlockSpec((NGATES * H, H), lambda t, j: (0, 0)),
            pl.BlockSpec((NGATES * H, H), lambda t, j: (0, 0)),
            pl.BlockSpec((NGATES * H, 1), lambda t, j: (0, 0)),
            pl.BlockSpec((NGATES * SW, SW), lambda t, j: (0, 0)),
            pl.BlockSpec((NGATES * SW, H), lambda t, j: (0, 0)),
            pl.BlockSpec((NGATES * SW, 2), lambda t, j: (0, 0)),
            pl.BlockSpec((NGATES * SW, 1), lambda t, j: (0, 0)),
            pl.BlockSpec((NGATES * SW, 1), lambda t, j: (0, 0)),
        ],
        out_specs=pl.BlockSpec((H, NB), lambda t, j: (0, 0)),
        out_shape=jax.ShapeDtypeStruct((H, NB), jnp.float32),
        scratch_shapes=[
            pltpu.VMEM((H + 8, TAP), BF),
            pltpu.VMEM((H, TAP), jnp.float32),
            pltpu.VMEM((SW, NB), jnp.float32),
            pltpu.VMEM((SW, NB), jnp.float32),
            pltpu.VMEM((H + 8, NB), jnp.float32),
            pltpu.VMEM((NGATES * H, NB), BF),
        ],
    )(afT, ab_row, ab_col, xhT, xt3, WxT.astype(BF), WrT.astype(BF), WlgT,
      gb0T, Mb0T, MbmT, MbxT, bxtT, browT)
    return outT.T


# speedup vs baseline: 27.8237x; 1.2985x over previous
import functools

import jax
import jax.numpy as jnp
from jax.experimental import pallas as pl
from jax.experimental.pallas import tpu as pltpu

H = 32
NGATES = 4
GATES = ["i", "f", "c", "o"]
SMALL = ["agent_summ", "hideout_summ", "state_summ", "hideout", "timestep"]
BLK = {nt: i for i, nt in enumerate(SMALL)}
SW = H * len(SMALL)
SMALL_EDGES = [
    ("hideout", "hideout_summ"),
    ("hideout_summ", "state_summ"),
    ("agent_summ", "state_summ"),
    ("timestep", "state_summ"),
    ("hideout_summ", "hideout"),
    ("state_summ", "hideout_summ"),
    ("state_summ", "agent_summ"),
    ("state_summ", "timestep"),
]


def _ek(s, d):
    return s + "__" + d


def _pack_weights(params):
    WxT = jnp.concatenate(
        [params[g]["W"]["agent"] for g in GATES], axis=1).T
    WrT = jnp.concatenate(
        [params[g]["conv"][_ek("agent_summ", "agent")]["Wr"] for g in GATES],
        axis=1).T
    WlgT = jnp.concatenate(
        [params[g]["conv"][_ek("agent_summ", "agent")]["Wl"] for g in GATES],
        axis=1).T
    gb0T = jnp.concatenate(
        [
            (
                params[g]["conv"][_ek("agent_summ", "agent")]["bl"][None, :]
                + params[g]["b"]["agent"]
            )
            for g in GATES
        ],
        axis=1,
    ).T

    ZH = jnp.zeros((H, H), jnp.float32)
    rows = []
    for s in SMALL:
        row = []
        for g in GATES:
            conv = params[g]["conv"]
            for d in SMALL:
                acc = ZH
                for es, ed in SMALL_EDGES:
                    if ed != d:
                        continue
                    p = conv[_ek(es, ed)]
                    if es == s:
                        acc = acc + p["Wl"]
                    if ed == s:
                        acc = acc + p["Wr"]
                if d == "agent_summ" and s == "agent_summ":
                    acc = acc + conv[_ek("agent", "agent_summ")]["Wr"]
                row.append(acc)
        rows.append(jnp.concatenate(row, axis=1))
    Mb0 = jnp.concatenate(rows, axis=0)

    Z2 = jnp.zeros((2, H), jnp.float32)
    Z1 = jnp.zeros((1, H), jnp.float32)
    mbm, mbx, bxt_r, brow_r = [], [], [], []
    for g in GATES:
        conv = params[g]["conv"]
        for d in SMALL:
            mbm.append(conv[_ek("agent", "agent_summ")]["Wl"]
                       if d == "agent_summ" else ZH)
            mbx.append(params[g]["W"]["hideout"] if d == "hideout" else Z2)
            bxt_r.append(params[g]["W"]["timestep"]
                         if d == "timestep" else Z1)
            b = params[g]["b"][d]
            for es, ed in SMALL_EDGES:
                if ed == d:
                    b = b + conv[_ek(es, ed)]["bl"][None, :]
            if d == "agent_summ":
                b = b + conv[_ek("agent", "agent_summ")]["bl"][None, :]
            brow_r.append(b)
    Mbm = jnp.concatenate(mbm, axis=1)
    Mbx = jnp.concatenate(mbx, axis=1)
    bxt = jnp.concatenate(bxt_r, axis=1)
    brow = jnp.concatenate(brow_r, axis=1)
    return WxT, WrT, WlgT, gb0T, Mb0.T, Mbm.T, Mbx.T, bxt.T, brow.T


def _dot(a, b):
    return jnp.dot(a, b, preferred_element_type=jnp.float32)


def _sig(z):
    return 0.5 + 0.5 * jnp.tanh(0.5 * z)


BF = jnp.bfloat16


def _body(x_ref, abr_ref, abc_ref, xh_ref, xt_ref, Wx_ref, Wr_ref, Wlg_ref,
          gb0_ref, Mb0_ref, Mbm_ref, Mbx_ref, bxt_ref, brow_ref, out_ref,
          h_a, c_a, h_s, c_s, m_s, wall_s, *, TILE, NTILES, TA, SEQ, NB):
    t = pl.program_id(0)
    j = pl.program_id(1)
    cols = pl.ds(j * TILE, TILE)

    @pl.when(t == 0)
    def _zero_tile():
        h_a[:, cols] = jnp.concatenate(
            [jnp.zeros((H, TILE), BF), jnp.ones((8, TILE), BF)], axis=0)
        c_a[:, cols] = jnp.zeros((H, TILE), jnp.float32)

    @pl.when((t == 0) & (j == 0))
    def _zero_small():
        h_s[...] = jnp.zeros((SW, NB), jnp.float32)
        c_s[...] = jnp.zeros((SW, NB), jnp.float32)

    @pl.when(j == 0)
    def _start_step():
        gb = (_dot(Wlg_ref[...], h_s[0:H, :]) + gb0_ref[...]).astype(BF)
        wall_s[...] = jnp.concatenate(
            [Wx_ref[...], Wr_ref[...], jnp.zeros((NGATES * H, 8), BF), gb],
            axis=1)
        m_s[...] = jnp.zeros((H + 8, NB), jnp.float32)

    ab_r = abr_ref[0]
    ab_c = abc_ref[0]
    oT = (jax.lax.broadcasted_iota(jnp.int32, (NB, TILE), 0) == ab_r
          ).astype(BF)
    oh = (jax.lax.broadcasted_iota(jnp.int32, (TILE, NB), 1) == ab_c
          ).astype(BF)

    h1_prev = h_a[:, cols]
    c_prev = c_a[:, cols]
    m_s[...] += _dot(h1_prev, oh)

    x = x_ref[0]
    fused_in = jnp.concatenate([x, h1_prev, oT], axis=0)
    pre = _dot(wall_s[...], fused_in)
    ig = _sig(pre[0:H, :])
    fg = _sig(pre[H:2 * H, :])
    gg = jnp.tanh(pre[2 * H:3 * H, :])
    og = _sig(pre[3 * H:4 * H, :])
    c_new = fg * c_prev + ig * gg
    h_new = og * jnp.tanh(c_new)
    valid = (j * TILE + jax.lax.broadcasted_iota(jnp.int32, (1, TILE), 1)) < TA
    h_a[0:H, cols] = jnp.where(valid, h_new, 0.0).astype(BF)
    c_a[:, cols] = jnp.where(valid, c_new, 0.0)

    @pl.when(j == NTILES - 1)
    def _small_step():
        m = m_s[0:H, :] / jnp.maximum(m_s[H:H + 1, :], 1.0)
        pre_s = (_dot(Mb0_ref[...], h_s[...]) + _dot(Mbm_ref[...], m)
                 + _dot(Mbx_ref[...], xh_ref[...])
                 + bxt_ref[...] * xt_ref[0]
                 + brow_ref[...])
        i_s = _sig(pre_s[0:SW, :])
        f_s = _sig(pre_s[SW:2 * SW, :])
        g_s = jnp.tanh(pre_s[2 * SW:3 * SW, :])
        o_s = _sig(pre_s[3 * SW:4 * SW, :])
        c_ns = f_s * c_s[...] + i_s * g_s
        h_ns = o_s * jnp.tanh(c_ns)
        c_s[...] = c_ns
        h_s[...] = h_ns

        @pl.when(t == SEQ - 1)
        def _emit():
            out_ref[...] = h_ns[2 * H:3 * H, :]


def kernel(agent_feats, hideout_obs, timestep_obs, params, agent_batch):
    SEQ, TA, F = agent_feats.shape
    NB = hideout_obs.shape[0]
    TILE = 4096
    NTILES = max(1, -(-TA // TILE))
    TAP = NTILES * TILE

    WxT, WrT, WlgT, gb0T, Mb0T, MbmT, MbxT, bxtT, browT = _pack_weights(params)

    ab = agent_batch.astype(jnp.int32)
    abp = jnp.pad(ab, (0, TAP - TA), constant_values=127)
    ab_row = abp.reshape(NTILES, 1, TILE)
    ab_col = abp.reshape(NTILES, TILE, 1)
    afT = agent_feats.transpose(0, 2, 1).astype(BF)
    xhT = hideout_obs.T
    xt3 = timestep_obs.T.reshape(SEQ, 1, NB)

    body = functools.partial(_body, TILE=TILE, NTILES=NTILES, TA=TA, SEQ=SEQ,
                             NB=NB)
    grid = (SEQ, NTILES)
    outT = pl.pallas_call(
        body,
        grid=grid,
        in_specs=[
            pl.BlockSpec((1, F, TILE), lambda t, j: (t, 0, j)),
            pl.BlockSpec((1, 1, TILE), lambda t, j: (j, 0, 0)),
            pl.BlockSpec((1, TILE, 1), lambda t, j: (j, 0, 0)),
            pl.BlockSpec((2, NB), lambda t, j: (0, 0)),
            pl.BlockSpec((1, 1, NB), lambda t, j: (t, 0, 0)),
            pl.BlockSpec((NGATES * H, F), lambda t, j: (0, 0)),
            pl.BlockSpec((NGATES * H, H), lambda t, j: (0, 0)),
            pl.BlockSpec((NGATES * H, H), lambda t, j: (0, 0)),
            pl.BlockSpec((NGATES * H, 1), lambda t, j: (0, 0)),
            pl.BlockSpec((NGATES * SW, SW), lambda t, j: (0, 0)),
            pl.BlockSpec((NGATES * SW, H), lambda t, j: (0, 0)),
            pl.BlockSpec((NGATES * SW, 2), lambda t, j: (0, 0)),
            pl.BlockSpec((NGATES * SW, 1), lambda t, j: (0, 0)),
            pl.BlockSpec((NGATES * SW, 1), lambda t, j: (0, 0)),
        ],
        out_specs=pl.BlockSpec((H, NB), lambda t, j: (0, 0)),
        out_shape=jax.ShapeDtypeStruct((H, NB), jnp.float32),
        scratch_shapes=[
            pltpu.VMEM((H + 8, TAP), BF),
            pltpu.VMEM((H, TAP), jnp.float32),
            pltpu.VMEM((SW, NB), jnp.float32),
            pltpu.VMEM((SW, NB), jnp.float32),
            pltpu.VMEM((H + 8, NB), jnp.float32),
            pltpu.VMEM((NGATES * H, 112), BF),
        ],
    )(afT, ab_row, ab_col, xhT, xt3, WxT.astype(BF), WrT.astype(BF), WlgT,
      gb0T, Mb0T, MbmT, MbxT, bxtT, browT)
    return outT.T


# speedup vs baseline: 47.1005x; 1.6928x over previous
import functools

import jax
import jax.numpy as jnp
from jax.experimental import pallas as pl
from jax.experimental.pallas import tpu as pltpu

H = 32
NGATES = 4
GATES = ["i", "f", "c", "o"]
SMALL = ["agent_summ", "hideout_summ", "state_summ", "hideout", "timestep"]
BLK = {nt: i for i, nt in enumerate(SMALL)}
SW = H * len(SMALL)
SMALL_EDGES = [
    ("hideout", "hideout_summ"),
    ("hideout_summ", "state_summ"),
    ("agent_summ", "state_summ"),
    ("timestep", "state_summ"),
    ("hideout_summ", "hideout"),
    ("state_summ", "hideout_summ"),
    ("state_summ", "agent_summ"),
    ("state_summ", "timestep"),
]


def _ek(s, d):
    return s + "__" + d


def _pack_weights(params):
    WxT = jnp.concatenate(
        [params[g]["W"]["agent"] for g in GATES], axis=1).T
    WrT = jnp.concatenate(
        [params[g]["conv"][_ek("agent_summ", "agent")]["Wr"] for g in GATES],
        axis=1).T
    WlgT = jnp.concatenate(
        [params[g]["conv"][_ek("agent_summ", "agent")]["Wl"] for g in GATES],
        axis=1).T
    gb0T = jnp.concatenate(
        [
            (
                params[g]["conv"][_ek("agent_summ", "agent")]["bl"][None, :]
                + params[g]["b"]["agent"]
            )
            for g in GATES
        ],
        axis=1,
    ).T

    ZH = jnp.zeros((H, H), jnp.float32)
    rows = []
    for s in SMALL:
        row = []
        for g in GATES:
            conv = params[g]["conv"]
            for d in SMALL:
                acc = ZH
                for es, ed in SMALL_EDGES:
                    if ed != d:
                        continue
                    p = conv[_ek(es, ed)]
                    if es == s:
                        acc = acc + p["Wl"]
                    if ed == s:
                        acc = acc + p["Wr"]
                if d == "agent_summ" and s == "agent_summ":
                    acc = acc + conv[_ek("agent", "agent_summ")]["Wr"]
                row.append(acc)
        rows.append(jnp.concatenate(row, axis=1))
    Mb0 = jnp.concatenate(rows, axis=0)

    Z2 = jnp.zeros((2, H), jnp.float32)
    Z1 = jnp.zeros((1, H), jnp.float32)
    mbm, mbx, bxt_r, brow_r = [], [], [], []
    for g in GATES:
        conv = params[g]["conv"]
        for d in SMALL:
            mbm.append(conv[_ek("agent", "agent_summ")]["Wl"]
                       if d == "agent_summ" else ZH)
            mbx.append(params[g]["W"]["hideout"] if d == "hideout" else Z2)
            bxt_r.append(params[g]["W"]["timestep"]
                         if d == "timestep" else Z1)
            b = params[g]["b"][d]
            for es, ed in SMALL_EDGES:
                if ed == d:
                    b = b + conv[_ek(es, ed)]["bl"][None, :]
            if d == "agent_summ":
                b = b + conv[_ek("agent", "agent_summ")]["bl"][None, :]
            brow_r.append(b)
    Mbm = jnp.concatenate(mbm, axis=1)
    Mbx = jnp.concatenate(mbx, axis=1)
    bxt = jnp.concatenate(bxt_r, axis=1)
    brow = jnp.concatenate(brow_r, axis=1)

    sc_a = jnp.repeat(jnp.array([0.5, 0.5, 1.0, 0.5], jnp.float32),
                      H)[:, None]
    sc_s = jnp.repeat(jnp.array([0.5, 0.5, 1.0, 0.5], jnp.float32),
                      SW)[:, None]
    return (WxT * sc_a, WrT * sc_a, WlgT * sc_a, gb0T * sc_a,
            Mb0.T * sc_s, Mbm.T * sc_s, Mbx.T * sc_s, bxt.T * sc_s,
            brow.T * sc_s)


def _dot(a, b):
    return jnp.dot(a, b, preferred_element_type=jnp.float32)


def _sig(z):
    return 0.5 + 0.5 * jnp.tanh(z)


BF = jnp.bfloat16


def _body(x_ref, abr_ref, xh_ref, xt_ref, Wx_ref, Wr_ref, Wlg_ref,
          gb0_ref, Mb0_ref, Mbm_ref, Mbx_ref, bxt_ref, brow_ref, out_ref,
          h_a, c_a, h_s, c_s, m_s, wall_s, *, TILE, NTILES, TA, SEQ, NB):
    t = pl.program_id(0)
    j = pl.program_id(1)
    cols = pl.ds(j * TILE, TILE)

    @pl.when(t == 0)
    def _zero_tile():
        h_a[:, cols] = jnp.concatenate(
            [jnp.zeros((H, TILE), BF), jnp.ones((8, TILE), BF)], axis=0)
        c_a[:, cols] = jnp.zeros((H, TILE), jnp.float32)

    @pl.when((t == 0) & (j == 0))
    def _zero_small():
        h_s[...] = jnp.zeros((SW, NB), jnp.float32)
        c_s[...] = jnp.zeros((SW, NB), jnp.float32)

    @pl.when(j == 0)
    def _start_step():
        gb = (_dot(Wlg_ref[...], h_s[0:H, :]) + gb0_ref[...]).astype(BF)
        wall_s[...] = jnp.concatenate(
            [Wx_ref[...], Wr_ref[...], jnp.zeros((NGATES * H, 8), BF), gb],
            axis=1)
        m_s[...] = jnp.zeros((H + 8, NB), jnp.float32)

    ab_r = abr_ref[0]
    oT = (jax.lax.broadcasted_iota(jnp.int32, (NB, TILE), 0) == ab_r
          ).astype(BF)

    h1_prev = h_a[:, cols]
    c_prev = c_a[:, cols]
    m_s[...] += jax.lax.dot_general(
        h1_prev, oT, (((1,), (1,)), ((), ())),
        preferred_element_type=jnp.float32)

    x = x_ref[0]
    fused_in = jnp.concatenate([x, h1_prev, oT], axis=0)
    pre = _dot(wall_s[...], fused_in)
    ig = _sig(pre[0:H, :])
    fg = _sig(pre[H:2 * H, :])
    gg = jnp.tanh(pre[2 * H:3 * H, :])
    og = _sig(pre[3 * H:4 * H, :])
    c_new = fg * c_prev + ig * gg
    h_new = og * jnp.tanh(c_new)
    valid = (j * TILE + jax.lax.broadcasted_iota(jnp.int32, (1, TILE), 1)) < TA
    h_a[0:H, cols] = jnp.where(valid, h_new, 0.0).astype(BF)
    c_a[:, cols] = jnp.where(valid, c_new, 0.0)

    @pl.when(j == NTILES - 1)
    def _small_step():
        m = m_s[0:H, :] / jnp.maximum(m_s[H:H + 1, :], 1.0)
        pre_s = (_dot(Mb0_ref[...], h_s[...]) + _dot(Mbm_ref[...], m)
                 + _dot(Mbx_ref[...], xh_ref[...])
                 + bxt_ref[...] * xt_ref[0]
                 + brow_ref[...])
        i_s = _sig(pre_s[0:SW, :])
        f_s = _sig(pre_s[SW:2 * SW, :])
        g_s = jnp.tanh(pre_s[2 * SW:3 * SW, :])
        o_s = _sig(pre_s[3 * SW:4 * SW, :])
        c_ns = f_s * c_s[...] + i_s * g_s
        h_ns = o_s * jnp.tanh(c_ns)
        c_s[...] = c_ns
        h_s[...] = h_ns

        @pl.when(t == SEQ - 1)
        def _emit():
            out_ref[...] = h_ns[2 * H:3 * H, :]


def kernel(agent_feats, hideout_obs, timestep_obs, params, agent_batch):
    SEQ, TA, F = agent_feats.shape
    NB = hideout_obs.shape[0]
    TILE = 4096
    NTILES = max(1, -(-TA // TILE))
    TAP = NTILES * TILE

    WxT, WrT, WlgT, gb0T, Mb0T, MbmT, MbxT, bxtT, browT = _pack_weights(params)

    ab = agent_batch.astype(jnp.int32)
    abp = jnp.pad(ab, (0, TAP - TA), constant_values=127)
    ab_row = abp.reshape(NTILES, 1, TILE)
    afT = agent_feats.transpose(0, 2, 1).astype(BF)
    xhT = hideout_obs.T
    xt3 = timestep_obs.T.reshape(SEQ, 1, NB)

    body = functools.partial(_body, TILE=TILE, NTILES=NTILES, TA=TA, SEQ=SEQ,
                             NB=NB)
    grid = (SEQ, NTILES)
    outT = pl.pallas_call(
        body,
        grid=grid,
        in_specs=[
            pl.BlockSpec((1, F, TILE), lambda t, j: (t, 0, j)),
            pl.BlockSpec((1, 1, TILE), lambda t, j: (j, 0, 0)),
            pl.BlockSpec((2, NB), lambda t, j: (0, 0)),
            pl.BlockSpec((1, 1, NB), lambda t, j: (t, 0, 0)),
            pl.BlockSpec((NGATES * H, F), lambda t, j: (0, 0)),
            pl.BlockSpec((NGATES * H, H), lambda t, j: (0, 0)),
            pl.BlockSpec((NGATES * H, H), lambda t, j: (0, 0)),
            pl.BlockSpec((NGATES * H, 1), lambda t, j: (0, 0)),
            pl.BlockSpec((NGATES * SW, SW), lambda t, j: (0, 0)),
            pl.BlockSpec((NGATES * SW, H), lambda t, j: (0, 0)),
            pl.BlockSpec((NGATES * SW, 2), lambda t, j: (0, 0)),
            pl.BlockSpec((NGATES * SW, 1), lambda t, j: (0, 0)),
            pl.BlockSpec((NGATES * SW, 1), lambda t, j: (0, 0)),
        ],
        out_specs=pl.BlockSpec((H, NB), lambda t, j: (0, 0)),
        out_shape=jax.ShapeDtypeStruct((H, NB), jnp.float32),
        scratch_shapes=[
            pltpu.VMEM((H + 8, TAP), BF),
            pltpu.VMEM((H, TAP), jnp.float32),
            pltpu.VMEM((SW, NB), jnp.float32),
            pltpu.VMEM((SW, NB), jnp.float32),
            pltpu.VMEM((H + 8, NB), jnp.float32),
            pltpu.VMEM((NGATES * H, 112), BF),
        ],
    )(afT, ab_row, xhT, xt3, WxT.astype(BF), WrT.astype(BF), WlgT,
      gb0T, Mb0T, MbmT, MbxT, bxtT, browT)
    return outT.T
